# initial kernel scaffold (unmeasured)
import jax
import jax.numpy as jnp
from jax import lax
from jax.experimental import pallas as pl
from jax.experimental.pallas import tpu as pltpu

N_DEV = 32


def kernel(x, w_mat):
    m, k_per = x.shape
    _, n = w_mat.shape
    m_per = m // N_DEV
    n_hops = N_DEV - 1

    def body(x_ref, w_ref, out_ref, part_ref, comm_ref, send_sems, recv_sems):
        my = lax.axis_index("i")
        left = lax.rem(my - 1 + N_DEV, N_DEV)
        right = lax.rem(my + 1, N_DEV)

        barrier_sem = pltpu.get_barrier_semaphore()
        for nbr in (left, right):
            pl.semaphore_signal(
                barrier_sem, inc=1,
                device_id=(nbr,), device_id_type=pl.DeviceIdType.MESH,
            )
        pl.semaphore_wait(barrier_sem, 2)

        part_ref[:, :] = jnp.dot(
            x_ref[:, :], w_ref[:, :], preferred_element_type=jnp.float32
        )

        def block(c):
            return part_ref[pl.ds(c * m_per, m_per), :]

        for h in range(n_hops):
            if h == 0:
                src = block(lax.rem(my - 1 + N_DEV, N_DEV))
            else:
                src = comm_ref.at[h - 1]
            rdma = pltpu.make_async_remote_copy(
                src_ref=src,
                dst_ref=comm_ref.at[h],
                send_sem=send_sems.at[h],
                recv_sem=recv_sems.at[h],
                device_id=(right,),
                device_id_type=pl.DeviceIdType.MESH,
            )
            rdma.start()
            rdma.wait()

            c = lax.rem(my - 2 - h + 2 * N_DEV, N_DEV)
            if h < n_hops - 1:
                comm_ref[h, :, :] = comm_ref[h, :, :] + block(c)
            else:
                y = comm_ref[h, :, :] + block(c)
                y = jnp.clip(y, -60.0, 60.0)
                out_ref[:, :] = y / (1.0 + jnp.exp(-y)) * jnp.exp(
                    jnp.zeros((), jnp.float32)
                )

    def body_fixed(x_ref, w_ref, out_ref, part_ref, comm_ref, send_sems,
                   recv_sems):
        return body(x_ref, w_ref, out_ref, part_ref, comm_ref, send_sems,
                    recv_sems)

    return pl.pallas_call(
        body_fixed,
        out_shape=jax.ShapeDtypeStruct((m_per, n), jnp.float32),
        in_specs=[
            pl.BlockSpec(memory_space=pltpu.VMEM),
            pl.BlockSpec(memory_space=pltpu.VMEM),
        ],
        out_specs=pl.BlockSpec(memory_space=pltpu.VMEM),
        scratch_shapes=[
            pltpu.VMEM((m, n), jnp.float32),
            pltpu.VMEM((n_hops, m_per, n), jnp.float32),
            pltpu.SemaphoreType.DMA((n_hops,)),
            pltpu.SemaphoreType.DMA((n_hops,)),
        ],
        compiler_params=pltpu.CompilerParams(collective_id=0),
    )(x, w_mat)


# baseline (device time: 242027 ns/iter reference)
import jax
import jax.numpy as jnp
from jax import lax
from jax.experimental import pallas as pl
from jax.experimental.pallas import tpu as pltpu

N_DEV = 32


def kernel(x, w_mat):
    m, k_per = x.shape
    _, n = w_mat.shape
    m_per = m // N_DEV
    n_hops = N_DEV - 1

    def body(x_ref, w_ref, out_ref, part_ref, comm_ref, send_sems, recv_sems):
        my = lax.axis_index("i")
        left = lax.rem(my - 1 + N_DEV, N_DEV)
        right = lax.rem(my + 1, N_DEV)

        barrier_sem = pltpu.get_barrier_semaphore()
        for nbr in (left, right):
            pl.semaphore_signal(
                barrier_sem, inc=1,
                device_id=(nbr,), device_id_type=pl.DeviceIdType.MESH,
            )
        pl.semaphore_wait(barrier_sem, 2)

        part_ref[:, :] = jnp.dot(
            x_ref[:, :], w_ref[:, :], preferred_element_type=jnp.float32
        )

        def block(c):
            return part_ref[pl.ds(c * m_per, m_per), :]

        for h in range(n_hops):
            if h == 0:
                c0 = lax.rem(my - 1 + N_DEV, N_DEV)
                src = part_ref.at[pl.ds(c0 * m_per, m_per), :]
            else:
                src = comm_ref.at[h - 1]
            rdma = pltpu.make_async_remote_copy(
                src_ref=src,
                dst_ref=comm_ref.at[h],
                send_sem=send_sems.at[h],
                recv_sem=recv_sems.at[h],
                device_id=(right,),
                device_id_type=pl.DeviceIdType.MESH,
            )
            rdma.start()
            rdma.wait()

            c = lax.rem(my - 2 - h + 2 * N_DEV, N_DEV)
            if h < n_hops - 1:
                comm_ref[h, :, :] = comm_ref[h, :, :] + block(c)
            else:
                y = comm_ref[h, :, :] + block(c)
                yc = jnp.clip(y, -60.0, 60.0)
                out_ref[:, :] = y / (1.0 + jnp.exp(-yc))

    return pl.pallas_call(
        body,
        out_shape=jax.ShapeDtypeStruct((m_per, n), jnp.float32),
        in_specs=[
            pl.BlockSpec(memory_space=pltpu.VMEM),
            pl.BlockSpec(memory_space=pltpu.VMEM),
        ],
        out_specs=pl.BlockSpec(memory_space=pltpu.VMEM),
        scratch_shapes=[
            pltpu.VMEM((m, n), jnp.float32),
            pltpu.VMEM((n_hops, m_per, n), jnp.float32),
            pltpu.SemaphoreType.DMA((n_hops,)),
            pltpu.SemaphoreType.DMA((n_hops,)),
        ],
        compiler_params=pltpu.CompilerParams(collective_id=0),
    )(x, w_mat)


# device time: 192628 ns/iter; 1.2564x vs baseline; 1.2564x over previous
import jax
import jax.numpy as jnp
from jax import lax
from jax.experimental import pallas as pl
from jax.experimental.pallas import tpu as pltpu

N_DEV = 32
SEG = 4


def kernel(x, w_mat):
    m, k_per = x.shape
    _, n = w_mat.shape
    m_per = m // N_DEV
    n_hops = N_DEV - 1
    half = n // 2
    segw = half // SEG

    def body(x_ref, w_ref, out_ref, part_ref, rbufs, send_sems, recv_sems):
        my = lax.axis_index("i")
        left = lax.rem(my - 1 + N_DEV, N_DEV)
        right = lax.rem(my + 1, N_DEV)

        part_ref[:, :] = jnp.dot(
            x_ref[:, :], w_ref[:, :], preferred_element_type=jnp.float32
        )

        barrier_sem = pltpu.get_barrier_semaphore()
        for nbr in (left, right):
            pl.semaphore_signal(
                barrier_sem, inc=1,
                device_id=(nbr,), device_id_type=pl.DeviceIdType.MESH,
            )
        pl.semaphore_wait(barrier_sem, 2)

        dir_dst = (right, left)
        dir_col0 = (0, half)

        def part_seg_ref(c, dirn, j):
            return part_ref.at[
                pl.ds(c * m_per, m_per),
                pl.ds(dir_col0[dirn] + j * segw, segw),
            ]

        def send_block(dirn, h):
            if dirn == 0:
                return lax.rem(my - 1 - h + 2 * N_DEV, N_DEV)
            return lax.rem(my + 1 + h, N_DEV)

        def recv_block(dirn, h):
            if dirn == 0:
                return lax.rem(my - 2 - h + 2 * N_DEV, N_DEV)
            return lax.rem(my + 2 + h, N_DEV)

        def make_rdma(dirn, h, j):
            if h == 0:
                src = part_seg_ref(send_block(dirn, 0), dirn, j)
            else:
                src = rbufs.at[dirn, h - 1, j]
            return pltpu.make_async_remote_copy(
                src_ref=src,
                dst_ref=rbufs.at[dirn, h, j],
                send_sem=send_sems.at[dirn, h, j],
                recv_sem=recv_sems.at[dirn, h, j],
                device_id=(dir_dst[dirn],),
                device_id_type=pl.DeviceIdType.MESH,
            )

        rdmas = {}
        for j in range(SEG):
            for dirn in (0, 1):
                r = make_rdma(dirn, 0, j)
                r.start()
                rdmas[(dirn, 0, j)] = r

        for h in range(n_hops):
            for j in range(SEG):
                for dirn in (0, 1):
                    rdmas[(dirn, h, j)].wait_recv()
                    c = recv_block(dirn, h)
                    if h < n_hops - 1:
                        rbufs[dirn, h, j] = (
                            rbufs[dirn, h, j]
                            + part_seg_ref(c, dirn, j)[:, :]
                        )
                        r = make_rdma(dirn, h + 1, j)
                        r.start()
                        rdmas[(dirn, h + 1, j)] = r
                    else:
                        y = rbufs[dirn, h, j] + part_seg_ref(c, dirn, j)[:, :]
                        yc = jnp.clip(y, -60.0, 60.0)
                        out_ref[:, pl.ds(dir_col0[dirn] + j * segw, segw)] = (
                            y / (1.0 + jnp.exp(-yc))
                        )

        for key in rdmas:
            rdmas[key].wait_send()

    return pl.pallas_call(
        body,
        out_shape=jax.ShapeDtypeStruct((m_per, n), jnp.float32),
        in_specs=[
            pl.BlockSpec(memory_space=pltpu.VMEM),
            pl.BlockSpec(memory_space=pltpu.VMEM),
        ],
        out_specs=pl.BlockSpec(memory_space=pltpu.VMEM),
        scratch_shapes=[
            pltpu.VMEM((m, n), jnp.float32),
            pltpu.VMEM((2, n_hops, SEG, m_per, segw), jnp.float32),
            pltpu.SemaphoreType.DMA((2, n_hops, SEG)),
            pltpu.SemaphoreType.DMA((2, n_hops, SEG)),
        ],
        compiler_params=pltpu.CompilerParams(collective_id=0),
    )(x, w_mat)


# device time: 189955 ns/iter; 1.2741x vs baseline; 1.0141x over previous
import jax
import jax.numpy as jnp
from jax import lax
from jax.experimental import pallas as pl
from jax.experimental.pallas import tpu as pltpu

N_DEV = 32
SEG = 1


def kernel(x, w_mat):
    m, k_per = x.shape
    _, n = w_mat.shape
    m_per = m // N_DEV
    n_hops = N_DEV - 1
    half = n // 2
    segw = half // SEG

    def body(x_ref, w_ref, out_ref, part_ref, rbufs, send_sems, recv_sems):
        my = lax.axis_index("i")
        left = lax.rem(my - 1 + N_DEV, N_DEV)
        right = lax.rem(my + 1, N_DEV)

        part_ref[:, :] = jnp.dot(
            x_ref[:, :], w_ref[:, :], preferred_element_type=jnp.float32
        )

        barrier_sem = pltpu.get_barrier_semaphore()
        for nbr in (left, right):
            pl.semaphore_signal(
                barrier_sem, inc=1,
                device_id=(nbr,), device_id_type=pl.DeviceIdType.MESH,
            )
        pl.semaphore_wait(barrier_sem, 2)

        dir_dst = (right, left)
        dir_col0 = (0, half)

        def part_seg_ref(c, dirn, j):
            return part_ref.at[
                pl.ds(c * m_per, m_per),
                pl.ds(dir_col0[dirn] + j * segw, segw),
            ]

        def send_block(dirn, h):
            if dirn == 0:
                return lax.rem(my - 1 - h + 2 * N_DEV, N_DEV)
            return lax.rem(my + 1 + h, N_DEV)

        def recv_block(dirn, h):
            if dirn == 0:
                return lax.rem(my - 2 - h + 2 * N_DEV, N_DEV)
            return lax.rem(my + 2 + h, N_DEV)

        def make_rdma(dirn, h, j):
            if h == 0:
                src = part_seg_ref(send_block(dirn, 0), dirn, j)
            else:
                src = rbufs.at[dirn, h - 1, j]
            return pltpu.make_async_remote_copy(
                src_ref=src,
                dst_ref=rbufs.at[dirn, h, j],
                send_sem=send_sems.at[dirn, h, j],
                recv_sem=recv_sems.at[dirn, h, j],
                device_id=(dir_dst[dirn],),
                device_id_type=pl.DeviceIdType.MESH,
            )

        rdmas = {}
        for j in range(SEG):
            for dirn in (0, 1):
                r = make_rdma(dirn, 0, j)
                r.start()
                rdmas[(dirn, 0, j)] = r

        for h in range(n_hops):
            for j in range(SEG):
                for dirn in (0, 1):
                    rdmas[(dirn, h, j)].wait_recv()
                    c = recv_block(dirn, h)
                    if h < n_hops - 1:
                        rbufs[dirn, h, j] = (
                            rbufs[dirn, h, j]
                            + part_seg_ref(c, dirn, j)[:, :]
                        )
                        r = make_rdma(dirn, h + 1, j)
                        r.start()
                        rdmas[(dirn, h + 1, j)] = r
                    else:
                        y = rbufs[dirn, h, j] + part_seg_ref(c, dirn, j)[:, :]
                        yc = jnp.clip(y, -60.0, 60.0)
                        out_ref[:, pl.ds(dir_col0[dirn] + j * segw, segw)] = (
                            y / (1.0 + jnp.exp(-yc))
                        )

        for key in rdmas:
            rdmas[key].wait_send()

    return pl.pallas_call(
        body,
        out_shape=jax.ShapeDtypeStruct((m_per, n), jnp.float32),
        in_specs=[
            pl.BlockSpec(memory_space=pltpu.VMEM),
            pl.BlockSpec(memory_space=pltpu.VMEM),
        ],
        out_specs=pl.BlockSpec(memory_space=pltpu.VMEM),
        scratch_shapes=[
            pltpu.VMEM((m, n), jnp.float32),
            pltpu.VMEM((2, n_hops, SEG, m_per, segw), jnp.float32),
            pltpu.SemaphoreType.DMA((2, n_hops, SEG)),
            pltpu.SemaphoreType.DMA((2, n_hops, SEG)),
        ],
        compiler_params=pltpu.CompilerParams(collective_id=0),
    )(x, w_mat)


# device time: 111201 ns/iter; 2.1765x vs baseline; 1.7082x over previous
import jax
import jax.numpy as jnp
from jax import lax
from jax.experimental import pallas as pl
from jax.experimental.pallas import tpu as pltpu

N_DEV = 32
PLANE = 8
NZ = 4


def kernel(x, w_mat):
    m, k_per = x.shape
    _, n = w_mat.shape
    m_per = m // N_DEV
    half = n // 2

    def body(x_ref, w_ref, out_ref, part_ref, rbuf1, rbuf2,
             s1_send, s1_recv, s2_send, s2_recv):
        p = lax.axis_index("i")
        z = lax.div(p, PLANE)
        q = lax.rem(p, PLANE)

        def q_to_r(qq):
            yy = lax.div(qq, 2)
            xx = lax.rem(qq + yy, 2)
            return jnp.where(xx == 1, 1 + yy, lax.rem(8 - yy, 8))

        def r_to_q(rr):
            xx = jnp.where((rr >= 1) & (rr <= 4), 1, 0)
            yy = jnp.where(xx == 1, rr - 1, lax.rem(8 - rr, 8))
            return 2 * yy + lax.rem(xx + yy, 2)

        r = q_to_r(q)

        succ = z * PLANE + r_to_q(lax.rem(r + 1, 8))
        pred = z * PLANE + r_to_q(lax.rem(r + 7, 8))
        up = lax.rem(z + 1, NZ) * PLANE + q
        down = lax.rem(z + 3, NZ) * PLANE + q

        part_ref[:, :] = jnp.dot(
            x_ref[:, :], w_ref[:, :], preferred_element_type=jnp.float32
        )

        barrier_sem = pltpu.get_barrier_semaphore()
        for nbr in (succ, pred, up, down):
            pl.semaphore_signal(
                barrier_sem, inc=1,
                device_id=(nbr,), device_id_type=pl.DeviceIdType.MESH,
            )
        pl.semaphore_wait(barrier_sem, 4)

        col0 = (0, half)
        sign = (-1, 1)
        dst1 = (succ, pred)
        dst2 = (up, down)

        def pref(b, dirn):
            return part_ref.at[pl.ds(b * m_per, m_per),
                               pl.ds(col0[dirn], half)]

        def pval(b, dirn):
            return part_ref[pl.ds(b * m_per, m_per),
                            pl.ds(col0[dirn], half)]

        rdmas = {}

        def p1_make(dirn, h, zb):
            if h == 0:
                rc = lax.rem(r + sign[dirn] + 16, 8)
                src = pref(zb * PLANE + r_to_q(rc), dirn)
            else:
                src = rbuf1.at[dirn, h - 1, zb]
            return pltpu.make_async_remote_copy(
                src_ref=src,
                dst_ref=rbuf1.at[dirn, h, zb],
                send_sem=s1_send.at[dirn, h, zb],
                recv_sem=s1_recv.at[dirn, h, zb],
                device_id=(dst1[dirn],),
                device_id_type=pl.DeviceIdType.MESH,
            )

        for zb in range(NZ):
            for dirn in (0, 1):
                rd = p1_make(dirn, 0, zb)
                rd.start()
                rdmas[(1, dirn, 0, zb)] = rd

        for h in range(PLANE - 1):
            for zb in range(NZ):
                for dirn in (0, 1):
                    rdmas[(1, dirn, h, zb)].wait_recv()
                    rc = lax.rem(r + sign[dirn] * (2 + h) + 32, 8)
                    b = zb * PLANE + r_to_q(rc)
                    rbuf1[dirn, h, zb] = rbuf1[dirn, h, zb] + pval(b, dirn)
                    if h < PLANE - 2:
                        rd = p1_make(dirn, h + 1, zb)
                        rd.start()
                        rdmas[(1, dirn, h + 1, zb)] = rd

        def p2_make(dirn, h):
            if h == 0:
                zc = lax.rem(z + sign[dirn] + 8, NZ)
                src = rbuf1.at[dirn, PLANE - 2, zc]
            else:
                src = rbuf2.at[dirn, h - 1]
            return pltpu.make_async_remote_copy(
                src_ref=src,
                dst_ref=rbuf2.at[dirn, h],
                send_sem=s2_send.at[dirn, h],
                recv_sem=s2_recv.at[dirn, h],
                device_id=(dst2[dirn],),
                device_id_type=pl.DeviceIdType.MESH,
            )

        for dirn in (0, 1):
            rd = p2_make(dirn, 0)
            rd.start()
            rdmas[(2, dirn, 0, 0)] = rd

        for h in range(NZ - 1):
            for dirn in (0, 1):
                rdmas[(2, dirn, h, 0)].wait_recv()
                zc = lax.rem(z + sign[dirn] * (2 + h) + 8, NZ)
                if h < NZ - 2:
                    rbuf2[dirn, h] = rbuf2[dirn, h] + rbuf1[dirn, PLANE - 2, zc]
                    rd = p2_make(dirn, h + 1)
                    rd.start()
                    rdmas[(2, dirn, h + 1, 0)] = rd
                else:
                    y = rbuf2[dirn, h] + rbuf1[dirn, PLANE - 2, z]
                    yc = jnp.clip(y, -60.0, 60.0)
                    out_ref[:, pl.ds(col0[dirn], half)] = (
                        y / (1.0 + jnp.exp(-yc))
                    )

        for key in rdmas:
            rdmas[key].wait_send()

    return pl.pallas_call(
        body,
        out_shape=jax.ShapeDtypeStruct((m_per, n), jnp.float32),
        in_specs=[
            pl.BlockSpec(memory_space=pltpu.VMEM),
            pl.BlockSpec(memory_space=pltpu.VMEM),
        ],
        out_specs=pl.BlockSpec(memory_space=pltpu.VMEM),
        scratch_shapes=[
            pltpu.VMEM((m, n), jnp.float32),
            pltpu.VMEM((2, PLANE - 1, NZ, m_per, half), jnp.float32),
            pltpu.VMEM((2, NZ - 1, m_per, half), jnp.float32),
            pltpu.SemaphoreType.DMA((2, PLANE - 1, NZ)),
            pltpu.SemaphoreType.DMA((2, PLANE - 1, NZ)),
            pltpu.SemaphoreType.DMA((2, NZ - 1)),
            pltpu.SemaphoreType.DMA((2, NZ - 1)),
        ],
        compiler_params=pltpu.CompilerParams(collective_id=0),
    )(x, w_mat)


# device time: 102691 ns/iter; 2.3568x vs baseline; 1.0829x over previous
import jax
import jax.numpy as jnp
from jax import lax
from jax.experimental import pallas as pl
from jax.experimental.pallas import tpu as pltpu

N_DEV = 32
PLANE = 8
NZ = 4


def kernel(x, w_mat):
    m, k_per = x.shape
    _, n = w_mat.shape
    m_per = m // N_DEV
    half = n // 2

    def body(x_ref, w_ref, out_ref, part_ref, rbuf1, rbuf2,
             s1_send, s1_recv, s2_send, s2_recv):
        p = lax.axis_index("i")
        z = lax.div(p, PLANE)
        q = lax.rem(p, PLANE)

        def q_to_r(qq):
            yy = lax.div(qq, 2)
            xx = lax.rem(qq + yy, 2)
            return jnp.where(xx == 1, 1 + yy, lax.rem(8 - yy, 8))

        def r_to_q(rr):
            xx = jnp.where((rr >= 1) & (rr <= 4), 1, 0)
            yy = jnp.where(xx == 1, rr - 1, lax.rem(8 - rr, 8))
            return 2 * yy + lax.rem(xx + yy, 2)

        r = q_to_r(q)

        succ = z * PLANE + r_to_q(lax.rem(r + 1, 8))
        pred = z * PLANE + r_to_q(lax.rem(r + 7, 8))
        up = lax.rem(z + 1, NZ) * PLANE + q
        down = lax.rem(z + 3, NZ) * PLANE + q

        part_ref[:, :] = jnp.dot(
            x_ref[:, :], w_ref[:, :], preferred_element_type=jnp.float32
        )

        barrier_sem = pltpu.get_barrier_semaphore()
        for nbr in (succ, pred, up, down):
            pl.semaphore_signal(
                barrier_sem, inc=1,
                device_id=(nbr,), device_id_type=pl.DeviceIdType.MESH,
            )
        pl.semaphore_wait(barrier_sem, 4)

        col0 = (0, half)
        sign = (-1, 1)
        dst1 = (succ, pred)
        dst2 = (up, down)

        def pref(b, dirn):
            return part_ref.at[pl.ds(b * m_per, m_per),
                               pl.ds(col0[dirn], half)]

        def pval(b, dirn):
            return part_ref[pl.ds(b * m_per, m_per),
                            pl.ds(col0[dirn], half)]

        rdmas = {}

        def zb_of(dirn, g):
            if dirn == 0:
                return lax.rem(z + 3 - g + NZ, NZ)
            return lax.rem(z + 1 + g, NZ)

        def p1_make(dirn, h, g):
            if h == 0:
                rc = lax.rem(r + sign[dirn] + 16, 8)
                src = pref(zb_of(dirn, g) * PLANE + r_to_q(rc), dirn)
            else:
                src = rbuf1.at[dirn, h - 1, g]
            return pltpu.make_async_remote_copy(
                src_ref=src,
                dst_ref=rbuf1.at[dirn, h, g],
                send_sem=s1_send.at[dirn, h, g],
                recv_sem=s1_recv.at[dirn, h, g],
                device_id=(dst1[dirn],),
                device_id_type=pl.DeviceIdType.MESH,
            )

        def p2_make(dirn, h):
            if h == 0:
                src = rbuf1.at[dirn, PLANE - 2, 0]
            else:
                src = rbuf2.at[dirn, h - 1]
            return pltpu.make_async_remote_copy(
                src_ref=src,
                dst_ref=rbuf2.at[dirn, h],
                send_sem=s2_send.at[dirn, h],
                recv_sem=s2_recv.at[dirn, h],
                device_id=(dst2[dirn],),
                device_id_type=pl.DeviceIdType.MESH,
            )

        for g in range(NZ):
            for dirn in (0, 1):
                rd = p1_make(dirn, 0, g)
                rd.start()
                rdmas[(1, dirn, 0, g)] = rd

        for h in range(PLANE - 1):
            for g in range(NZ):
                for dirn in (0, 1):
                    rdmas[(1, dirn, h, g)].wait_recv()
                    rc = lax.rem(r + sign[dirn] * (2 + h) + 32, 8)
                    b = zb_of(dirn, g) * PLANE + r_to_q(rc)
                    rbuf1[dirn, h, g] = rbuf1[dirn, h, g] + pval(b, dirn)
                    if h < PLANE - 2:
                        rd = p1_make(dirn, h + 1, g)
                        rd.start()
                        rdmas[(1, dirn, h + 1, g)] = rd
                    else:
                        if g == 0:
                            rd = p2_make(dirn, 0)
                            rd.start()
                            rdmas[(2, dirn, 0, 0)] = rd
                        elif g < NZ - 1:
                            rdmas[(2, dirn, g - 1, 0)].wait_recv()
                            rbuf2[dirn, g - 1] = (
                                rbuf2[dirn, g - 1] + rbuf1[dirn, PLANE - 2, g]
                            )
                            rd = p2_make(dirn, g)
                            rd.start()
                            rdmas[(2, dirn, g, 0)] = rd
                        else:
                            rdmas[(2, dirn, g - 1, 0)].wait_recv()
                            y = (
                                rbuf2[dirn, g - 1]
                                + rbuf1[dirn, PLANE - 2, g]
                            )
                            yc = jnp.clip(y, -60.0, 60.0)
                            out_ref[:, pl.ds(col0[dirn], half)] = (
                                y / (1.0 + jnp.exp(-yc))
                            )

        for key in rdmas:
            rdmas[key].wait_send()

    return pl.pallas_call(
        body,
        out_shape=jax.ShapeDtypeStruct((m_per, n), jnp.float32),
        in_specs=[
            pl.BlockSpec(memory_space=pltpu.VMEM),
            pl.BlockSpec(memory_space=pltpu.VMEM),
        ],
        out_specs=pl.BlockSpec(memory_space=pltpu.VMEM),
        scratch_shapes=[
            pltpu.VMEM((m, n), jnp.float32),
            pltpu.VMEM((2, PLANE - 1, NZ, m_per, half), jnp.float32),
            pltpu.VMEM((2, NZ - 1, m_per, half), jnp.float32),
            pltpu.SemaphoreType.DMA((2, PLANE - 1, NZ)),
            pltpu.SemaphoreType.DMA((2, PLANE - 1, NZ)),
            pltpu.SemaphoreType.DMA((2, NZ - 1)),
            pltpu.SemaphoreType.DMA((2, NZ - 1)),
        ],
        compiler_params=pltpu.CompilerParams(collective_id=0),
    )(x, w_mat)
